# baseline (speedup 1.0000x reference)
import functools

import jax
import jax.numpy as jnp
from jax import lax
from jax.experimental import pallas as pl
from jax.experimental.pallas import tpu as pltpu
from jax.experimental.pallas import tpu_sc as plsc

_NE = 1024
_D = 64
_TB = 1024
_NC = 2
_NS = 16
_NW = _NC * _NS
_LOSS_SCALE = 1.25


def _dist_argmin_body(x_ref, e_ref, idx_ref, msum_ref):
    i = pl.program_id(0)
    xb = x_ref[0]
    e = e_ref[...]
    sim = lax.dot_general(xb, e, (((1,), (1,)), ((), ())),
                          preferred_element_type=jnp.float32)
    xn = jnp.sum(xb * xb, axis=1, keepdims=True)
    en = jnp.sum(e * e, axis=1)
    d = xn + en[None, :] - 2.0 * sim
    idx_ref[...] = jnp.argmin(d, axis=1).astype(jnp.int32)[None, None, :]
    partial = jnp.sum(jnp.min(d, axis=1)).reshape(1, 1, 1)

    @pl.when(i == 0)
    def _():
        msum_ref[...] = jnp.zeros((1, 1, 1), jnp.float32)

    msum_ref[...] += partial


def _dist_argmin(x3, embeddings):
    nblk = x3.shape[0]
    n = nblk * _TB
    idx, msum = pl.pallas_call(
        _dist_argmin_body,
        grid=(nblk,),
        in_specs=[
            pl.BlockSpec((1, _TB, _D), lambda i: (i, 0, 0)),
            pl.BlockSpec((_NE, _D), lambda i: (0, 0)),
        ],
        out_specs=[
            pl.BlockSpec((1, 1, _TB), lambda i: (i, 0, 0)),
            pl.BlockSpec((1, 1, 1), lambda i: (0, 0, 0)),
        ],
        out_shape=[
            jax.ShapeDtypeStruct((nblk, 1, _TB), jnp.int32),
            jax.ShapeDtypeStruct((1, 1, 1), jnp.float32),
        ],
    )(x3, embeddings)
    return idx.reshape(-1), msum[0, 0, 0]


def _make_sc_gather(n_tokens, rows_per_b):
    bpw = n_tokens // _NW
    wpb = rows_per_b // bpw

    @functools.partial(
        pl.kernel,
        mesh=plsc.VectorSubcoreMesh(core_axis_name="c", subcore_axis_name="s"),
        out_type=jax.ShapeDtypeStruct((n_tokens // rows_per_b, rows_per_b, _D),
                                      jnp.float32),
        scratch_types=[
            pltpu.VMEM((bpw,), jnp.int32),
            pltpu.VMEM((bpw, _D), jnp.float32),
            pltpu.SemaphoreType.DMA,
        ],
        compiler_params=pltpu.CompilerParams(use_tc_tiling_on_sc=False),
    )
    def sc_gather(table_hbm, idx_hbm, out_hbm, idx_v, rows_v, sem):
        wid = lax.axis_index("s") * _NC + lax.axis_index("c")
        base = wid * bpw
        pltpu.sync_copy(idx_hbm.at[pl.ds(base, bpw)], idx_v)
        pltpu.async_copy(table_hbm.at[idx_v], rows_v, sem).wait()
        pltpu.sync_copy(rows_v,
                        out_hbm.at[wid // wpb, pl.ds((wid % wpb) * bpw, bpw)])

    return sc_gather


def kernel(x, embeddings):
    input_shape = x.shape
    x3 = x.reshape(-1, _TB, _D)
    nb = x3.shape[0]
    n = nb * _TB
    h = nb // 2
    idx_a, ms_a = _dist_argmin(x3[:h], embeddings)
    q_a = _make_sc_gather(h * _TB, input_shape[1])(embeddings, idx_a)
    idx_b, ms_b = _dist_argmin(x3[h:], embeddings)
    q_b = _make_sc_gather(h * _TB, input_shape[1])(embeddings, idx_b)
    out = jnp.concatenate([q_a, q_b], axis=0).reshape(input_shape)
    vq_loss = _LOSS_SCALE * (ms_a + ms_b) / (n * _D)
    return out, vq_loss


# speedup vs baseline: 1.0802x; 1.0802x over previous
import functools

import jax
import jax.numpy as jnp
from jax import lax
from jax.experimental import pallas as pl
from jax.experimental.pallas import tpu as pltpu
from jax.experimental.pallas import tpu_sc as plsc

_NE = 1024
_D = 64
_TB = 1024
_NC = 2
_NS = 16
_NW = _NC * _NS
_LOSS_SCALE = 1.25


def _dist_argmin_body(x_ref, e_ref, idx_ref, msum_ref):
    i = pl.program_id(0)
    xb = x_ref[0]
    e = e_ref[...]
    sim = lax.dot_general(xb, e, (((1,), (1,)), ((), ())),
                          preferred_element_type=jnp.float32)
    xn = jnp.sum(xb * xb, axis=1, keepdims=True)
    en = jnp.sum(e * e, axis=1)
    d = xn + en[None, :] - 2.0 * sim
    idx_ref[...] = jnp.argmin(d, axis=1).astype(jnp.int32)[None, None, :]
    partial = jnp.sum(jnp.min(d, axis=1)).reshape(1, 1, 1)

    @pl.when(i == 0)
    def _():
        msum_ref[...] = jnp.zeros((1, 1, 1), jnp.float32)

    msum_ref[...] += partial


def _dist_argmin(x3, embeddings):
    nblk = x3.shape[0]
    n = nblk * _TB
    idx, msum = pl.pallas_call(
        _dist_argmin_body,
        grid=(nblk,),
        in_specs=[
            pl.BlockSpec((1, _TB, _D), lambda i: (i, 0, 0)),
            pl.BlockSpec((_NE, _D), lambda i: (0, 0)),
        ],
        out_specs=[
            pl.BlockSpec((1, 1, _TB), lambda i: (i, 0, 0)),
            pl.BlockSpec((1, 1, 1), lambda i: (0, 0, 0)),
        ],
        out_shape=[
            jax.ShapeDtypeStruct((nblk, 1, _TB), jnp.int32),
            jax.ShapeDtypeStruct((1, 1, 1), jnp.float32),
        ],
    )(x3, embeddings)
    return idx.reshape(-1), msum[0, 0, 0]


def _make_sc_gather(n_tokens, rows_per_b):
    bpw = n_tokens // _NW
    wpb = rows_per_b // bpw

    @functools.partial(
        pl.kernel,
        mesh=plsc.VectorSubcoreMesh(core_axis_name="c", subcore_axis_name="s"),
        out_type=jax.ShapeDtypeStruct((n_tokens // rows_per_b, rows_per_b, _D),
                                      jnp.float32),
        scratch_types=[
            pltpu.VMEM((bpw,), jnp.int32),
            pltpu.VMEM((bpw, _D), jnp.float32),
            pltpu.SemaphoreType.DMA,
        ],
        compiler_params=pltpu.CompilerParams(use_tc_tiling_on_sc=False),
    )
    def sc_gather(table_hbm, idx_hbm, out_hbm, idx_v, rows_v, sem):
        wid = lax.axis_index("s") * _NC + lax.axis_index("c")
        base = wid * bpw
        pltpu.sync_copy(idx_hbm.at[pl.ds(base, bpw)], idx_v)
        pltpu.async_copy(table_hbm.at[idx_v], rows_v, sem).wait()
        pltpu.sync_copy(rows_v,
                        out_hbm.at[wid // wpb, pl.ds((wid % wpb) * bpw, bpw)])

    return sc_gather


def kernel(x, embeddings):
    input_shape = x.shape
    x3 = x.reshape(-1, _TB, _D)
    n = x3.shape[0] * _TB
    idx, min_sum = _dist_argmin(x3, embeddings)
    quant = _make_sc_gather(n, input_shape[1])(embeddings, idx)
    out = quant.reshape(input_shape)
    vq_loss = _LOSS_SCALE * min_sum / (n * _D)
    return out, vq_loss


# speedup vs baseline: 1.0935x; 1.0124x over previous
import functools

import jax
import jax.numpy as jnp
from jax import lax
from jax.experimental import pallas as pl
from jax.experimental.pallas import tpu as pltpu
from jax.experimental.pallas import tpu_sc as plsc

_NE = 1024
_D = 64
_TB = 1024
_NC = 2
_NS = 16
_NW = _NC * _NS
_LOSS_SCALE = 1.25


def _dist_argmin_body(x_ref, e_ref, idx_ref, msum_ref):
    i = pl.program_id(0)
    xb = x_ref[0]
    e = e_ref[...]
    sim2 = lax.dot_general(xb, e + e, (((1,), (1,)), ((), ())),
                           preferred_element_type=jnp.float32)
    xn = jnp.sum(xb * xb, axis=1, keepdims=True)
    en = jnp.sum(e * e, axis=1)
    d = xn + en[None, :] - sim2
    idx_ref[...] = jnp.argmin(d, axis=1).astype(jnp.int32)[None, None, :]
    partial = jnp.sum(jnp.min(d, axis=1)).reshape(1, 1, 1)

    @pl.when(i == 0)
    def _():
        msum_ref[...] = jnp.zeros((1, 1, 1), jnp.float32)

    msum_ref[...] += partial


def _dist_argmin(x3, embeddings):
    nblk = x3.shape[0]
    n = nblk * _TB
    idx, msum = pl.pallas_call(
        _dist_argmin_body,
        grid=(nblk,),
        in_specs=[
            pl.BlockSpec((1, _TB, _D), lambda i: (i, 0, 0)),
            pl.BlockSpec((_NE, _D), lambda i: (0, 0)),
        ],
        out_specs=[
            pl.BlockSpec((1, 1, _TB), lambda i: (i, 0, 0)),
            pl.BlockSpec((1, 1, 1), lambda i: (0, 0, 0)),
        ],
        out_shape=[
            jax.ShapeDtypeStruct((nblk, 1, _TB), jnp.int32),
            jax.ShapeDtypeStruct((1, 1, 1), jnp.float32),
        ],
    )(x3, embeddings)
    return idx.reshape(-1), msum[0, 0, 0]


def _make_sc_gather(n_tokens, rows_per_b):
    bpw = n_tokens // _NW
    wpb = rows_per_b // bpw

    @functools.partial(
        pl.kernel,
        mesh=plsc.VectorSubcoreMesh(core_axis_name="c", subcore_axis_name="s"),
        out_type=jax.ShapeDtypeStruct((n_tokens // rows_per_b, rows_per_b, _D),
                                      jnp.float32),
        scratch_types=[
            pltpu.VMEM((bpw,), jnp.int32),
            pltpu.VMEM((bpw, _D), jnp.float32),
            pltpu.SemaphoreType.DMA,
        ],
        compiler_params=pltpu.CompilerParams(use_tc_tiling_on_sc=False),
    )
    def sc_gather(table_hbm, idx_hbm, out_hbm, idx_v, rows_v, sem):
        wid = lax.axis_index("s") * _NC + lax.axis_index("c")
        base = wid * bpw
        pltpu.sync_copy(idx_hbm.at[pl.ds(base, bpw)], idx_v)
        pltpu.async_copy(table_hbm.at[idx_v], rows_v, sem).wait()
        pltpu.sync_copy(rows_v,
                        out_hbm.at[wid // wpb, pl.ds((wid % wpb) * bpw, bpw)])

    return sc_gather


def kernel(x, embeddings):
    input_shape = x.shape
    x3 = x.reshape(-1, _TB, _D)
    n = x3.shape[0] * _TB
    idx, min_sum = _dist_argmin(x3, embeddings)
    quant = _make_sc_gather(n, input_shape[1])(embeddings, idx)
    out = quant.reshape(input_shape)
    vq_loss = _LOSS_SCALE * min_sum / (n * _D)
    return out, vq_loss
